# baseline (speedup 1.0000x reference)
import functools

import jax
import jax.numpy as jnp
from jax import lax
from jax.experimental import pallas as pl
from jax.experimental.pallas import tpu as pltpu
from jax.experimental.pallas import tpu_sc as plsc

_BATCH = 4096
_HIST = 50
_D = 128
_NW = 32
_BPB = _BATCH // _NW
_NB = 4
_NCH = _BPB // _NB


def _sc_gather(idx_hbm, table_hbm, out_hbm, idx_v, rows_v, sem_g):
    wid = lax.axis_index("s") * 2 + lax.axis_index("c")
    b0 = wid * _BPB
    pltpu.sync_copy(idx_hbm.at[pl.ds(b0, _BPB)], idx_v)

    def g_copy(c, b, j):
        return pltpu.make_async_copy(
            table_hbm.at[idx_v.at[c * _NB + j]],
            rows_v.at[b, j],
            sem_g.at[b],
        )

    def g_start(c, b):
        for j in range(_NB):
            g_copy(c, b, j).start()

    def g_wait(c, b):
        for j in range(_NB):
            g_copy(c, b, j).wait()

    g_start(0, 0)

    def step(c, b):
        @pl.when(c + 1 < _NCH)
        def _():
            g_start(c + 1, 1 - b)
        g_wait(c, b)
        pltpu.sync_copy(rows_v.at[b], out_hbm.at[pl.ds(b0 + c * _NB, _NB)])
        return 1 - b

    lax.fori_loop(0, _NCH, step, 0)


@jax.jit
def _run(indices, embed_word):
    mesh = plsc.VectorSubcoreMesh(core_axis_name="c", subcore_axis_name="s")
    fn = pl.kernel(
        _sc_gather,
        out_type=jax.ShapeDtypeStruct((_BATCH, _HIST, _D), jnp.float32),
        mesh=mesh,
        scratch_types=[
            pltpu.VMEM((_BPB, _HIST), jnp.int32),
            pltpu.VMEM((2, _NB, _HIST, _D), jnp.float32),
            pltpu.SemaphoreType.DMA((2,)),
        ],
        compiler_params=pltpu.CompilerParams(use_tc_tiling_on_sc=True),
    )
    return fn(indices, embed_word)


def kernel(indices, embed_word):
    return _run(indices, embed_word)


# speedup vs baseline: 1.4590x; 1.4590x over previous
import functools

import jax
import jax.numpy as jnp
from jax import lax
from jax.experimental import pallas as pl
from jax.experimental.pallas import tpu as pltpu
from jax.experimental.pallas import tpu_sc as plsc

_BATCH = 4096
_HIST = 50
_D = 128
_B = _BATCH * _HIST
_NW = 32
_BPW = _B // _NW
_C = 256
_NCHUNK = _BPW // _C


def _sc_gather(idx_hbm, table_hbm, out_hbm, idx_v, rows_v, sem_g):
    wid = lax.axis_index("s") * 2 + lax.axis_index("c")
    pltpu.sync_copy(idx_hbm.at[wid], idx_v)
    base = wid * _BPW

    def g_copy(c, b):
        return pltpu.make_async_copy(
            table_hbm.at[idx_v.at[pl.ds(c * _C, _C)]],
            rows_v.at[b],
            sem_g.at[b],
        )

    g_copy(0, 0).start()

    def step(c, b):
        @pl.when(c + 1 < _NCHUNK)
        def _():
            g_copy(c + 1, 1 - b).start()
        g_copy(c, b).wait()
        pltpu.sync_copy(rows_v.at[b], out_hbm.at[pl.ds(base + c * _C, _C)])
        return 1 - b

    lax.fori_loop(0, _NCHUNK, step, 0)


@jax.jit
def _run(indices_t_flat, embed_word):
    mesh = plsc.VectorSubcoreMesh(core_axis_name="c", subcore_axis_name="s")
    fn = pl.kernel(
        _sc_gather,
        out_type=jax.ShapeDtypeStruct((_B, _D), jnp.float32),
        mesh=mesh,
        scratch_types=[
            pltpu.VMEM((_BPW,), jnp.int32),
            pltpu.VMEM((2, _C, _D), jnp.float32),
            pltpu.SemaphoreType.DMA((2,)),
        ],
    )
    return fn(indices_t_flat, embed_word)


def kernel(indices, embed_word):
    idx_t = indices.T.reshape(_NW, _BPW)
    out_t = _run(idx_t, embed_word)
    out_t = out_t.reshape(_HIST, _BATCH, _D)
    return out_t.transpose(1, 0, 2)
